# Initial kernel scaffold; baseline (speedup 1.0000x reference)
import jax
import jax.numpy as jnp
from jax.experimental import pallas as pl


def kernel(degrees, table1, table2):
    raise NotImplementedError("write your pallas kernel here")


# speedup vs baseline: 1.5757x; 1.5757x over previous
import functools

import jax
import jax.numpy as jnp
from jax import lax
from jax.experimental import pallas as pl
from jax.experimental.pallas import tpu as pltpu
from jax.experimental.pallas import tpu_sc as plsc

MAX_DEG = 512
ROWS = 2 * (MAX_DEG + 1)
D = 64
B = 64
N = 2048
P = B * N
NC, NS = 2, 16
NW = NC * NS
PW = P // NW
C = 512
G = C // 16
NCHUNK = PW // C


def _body(deg_hbm, tab_hbm, out_hbm, tab_v, idx0_v, idx1_v, out_v):
    wid = lax.axis_index("s") * NC + lax.axis_index("c")
    pltpu.sync_copy(tab_hbm, tab_v)
    base = wid * PW
    lane64 = lax.iota(jnp.int32, 16) * D

    for chunk in range(NCHUNK):
        off = base + chunk * C
        pltpu.sync_copy(deg_hbm.at[pl.ds(off, C)], idx0_v)
        pltpu.sync_copy(deg_hbm.at[pl.ds(P + off, C)], idx1_v)

        def group(g, _):
            i0 = idx0_v[pl.ds(g * 16, 16)]
            i1 = idx1_v[pl.ds(g * 16, 16)]
            a0 = jnp.clip(i0, 0, MAX_DEG) * D
            a1 = (jnp.clip(i1, 0, MAX_DEG) + (MAX_DEG + 1)) * D
            sbase = lane64 + g * (16 * D)
            for c in range(D):
                v0 = plsc.load_gather(tab_v, [a0 + c])
                v1 = plsc.load_gather(tab_v, [a1 + c])
                plsc.store_scatter(out_v, [sbase + c], v0 + v1)
            return 0

        lax.fori_loop(0, G, group, 0, unroll=False)
        pltpu.sync_copy(out_v, out_hbm.at[pl.ds(off * D, C * D)])


@jax.jit
def _run(deg_flat, tab_cat):
    mesh = plsc.VectorSubcoreMesh(core_axis_name="c", subcore_axis_name="s")
    f = pl.kernel(
        _body,
        out_type=jax.ShapeDtypeStruct((P * D,), jnp.float32),
        mesh=mesh,
        scratch_types=[
            pltpu.VMEM((ROWS * D,), jnp.float32),
            pltpu.VMEM((C,), jnp.int32),
            pltpu.VMEM((C,), jnp.int32),
            pltpu.VMEM((C * D,), jnp.float32),
        ],
        compiler_params=pltpu.CompilerParams(needs_layout_passes=False),
    )
    return f(deg_flat, tab_cat)


def kernel(degrees, table1, table2):
    tab = jnp.concatenate(
        [table1.at[0].set(0.0), table2.at[0].set(0.0)], axis=0
    ).reshape(ROWS * D)
    deg_flat = degrees.reshape(2 * P)
    out = _run(deg_flat, tab)
    return out.reshape(B, N, D)


# speedup vs baseline: 1.9546x; 1.2404x over previous
import functools

import jax
import jax.numpy as jnp
from jax import lax
from jax.experimental import pallas as pl
from jax.experimental.pallas import tpu as pltpu
from jax.experimental.pallas import tpu_sc as plsc

MAX_DEG = 512
ROWS = 2 * (MAX_DEG + 1)
D = 64
B = 64
N = 2048
P = B * N
NC, NS = 2, 16
NW = NC * NS
PW = P // NW
C = 512
G = C // 16
NCHUNK = PW // C


def _body(deg_hbm, tab_hbm, out_hbm, tab_v, idx0_v, idx1_v, out_v):
    wid = lax.axis_index("s") * NC + lax.axis_index("c")
    pltpu.sync_copy(tab_hbm, tab_v)
    base = wid * PW
    lane64 = lax.iota(jnp.int32, 16) * D

    def chunk_body(chunk, _):
        off = base + chunk * C
        pltpu.sync_copy(deg_hbm.at[pl.ds(off, C)], idx0_v)
        pltpu.sync_copy(deg_hbm.at[pl.ds(P + off, C)], idx1_v)

        @plsc.parallel_loop(0, G, unroll=2)
        def group(g):
            i0 = idx0_v[pl.ds(g * 16, 16)]
            i1 = idx1_v[pl.ds(g * 16, 16)]
            a0 = jnp.clip(i0, 0, MAX_DEG) * D
            a1 = (jnp.clip(i1, 0, MAX_DEG) + (MAX_DEG + 1)) * D
            sbase = lane64 + g * (16 * D)
            for c in range(D):
                v0 = plsc.load_gather(tab_v, [a0 + c])
                v1 = plsc.load_gather(tab_v, [a1 + c])
                plsc.store_scatter(out_v, [sbase + c], v0 + v1)

        pltpu.sync_copy(out_v, out_hbm.at[pl.ds(off * D, C * D)])
        return 0

    lax.fori_loop(0, NCHUNK, chunk_body, 0, unroll=False)


@jax.jit
def _run(deg_flat, tab_cat):
    mesh = plsc.VectorSubcoreMesh(core_axis_name="c", subcore_axis_name="s")
    f = pl.kernel(
        _body,
        out_type=jax.ShapeDtypeStruct((P * D,), jnp.float32),
        mesh=mesh,
        scratch_types=[
            pltpu.VMEM((ROWS * D,), jnp.float32),
            pltpu.VMEM((C,), jnp.int32),
            pltpu.VMEM((C,), jnp.int32),
            pltpu.VMEM((C * D,), jnp.float32),
        ],
        compiler_params=pltpu.CompilerParams(needs_layout_passes=False),
    )
    return f(deg_flat, tab_cat)


def kernel(degrees, table1, table2):
    tab = jnp.concatenate(
        [table1.at[0].set(0.0), table2.at[0].set(0.0)], axis=0
    ).reshape(ROWS * D)
    deg_flat = degrees.reshape(2 * P)
    out = _run(deg_flat, tab)
    return out.reshape(B, N, D)


# speedup vs baseline: 5.0311x; 2.5740x over previous
import functools

import jax
import jax.numpy as jnp
from jax import lax
from jax.experimental import pallas as pl
from jax.experimental.pallas import tpu as pltpu
from jax.experimental.pallas import tpu_sc as plsc

MAX_DEG = 512
ROWS = 2 * (MAX_DEG + 1)
D = 64
B = 64
N = 2048
P = B * N
NC, NS = 2, 16
NW = NC * NS
PW = P // NW
C = 512
G = C // 16
NCHUNK = PW // C


def _body(deg_hbm, tab_hbm, out_hbm, tab_v, idx0_v, idx1_v, out_v):
    wid = lax.axis_index("s") * NC + lax.axis_index("c")
    pltpu.sync_copy(tab_hbm, tab_v)
    base = wid * PW
    lane = lax.iota(jnp.int32, 16)
    lane64 = lane * D
    rot = [lax.rem(lane + c, jnp.int32(16)) for c in range(16)]

    def chunk_body(chunk, _):
        off = base + chunk * C
        pltpu.sync_copy(deg_hbm.at[pl.ds(off, C)], idx0_v)
        pltpu.sync_copy(deg_hbm.at[pl.ds(P + off, C)], idx1_v)

        @plsc.parallel_loop(0, G, unroll=2)
        def group(g):
            i0 = idx0_v[pl.ds(g * 16, 16)]
            i1 = idx1_v[pl.ds(g * 16, 16)]
            a0 = jnp.clip(i0, 0, MAX_DEG) * D
            a1 = (jnp.clip(i1, 0, MAX_DEG) + (MAX_DEG + 1)) * D
            sbase = lane64 + g * (16 * D)
            for c in range(16):
                a0c = a0 + rot[c]
                a1c = a1 + rot[c]
                sbc = sbase + rot[c]
                for k in range(0, D, 16):
                    v0 = plsc.load_gather(tab_v, [a0c + k])
                    v1 = plsc.load_gather(tab_v, [a1c + k])
                    plsc.store_scatter(out_v, [sbc + k], v0 + v1)

        pltpu.sync_copy(out_v, out_hbm.at[pl.ds(off * D, C * D)])
        return 0

    lax.fori_loop(0, NCHUNK, chunk_body, 0, unroll=False)


@jax.jit
def _run(deg_flat, tab_cat):
    mesh = plsc.VectorSubcoreMesh(core_axis_name="c", subcore_axis_name="s")
    f = pl.kernel(
        _body,
        out_type=jax.ShapeDtypeStruct((P * D,), jnp.float32),
        mesh=mesh,
        scratch_types=[
            pltpu.VMEM((ROWS * D,), jnp.float32),
            pltpu.VMEM((C,), jnp.int32),
            pltpu.VMEM((C,), jnp.int32),
            pltpu.VMEM((C * D,), jnp.float32),
        ],
        compiler_params=pltpu.CompilerParams(needs_layout_passes=False),
    )
    return f(deg_flat, tab_cat)


def kernel(degrees, table1, table2):
    tab = jnp.concatenate(
        [table1.at[0].set(0.0), table2.at[0].set(0.0)], axis=0
    ).reshape(ROWS * D)
    deg_flat = degrees.reshape(2 * P)
    out = _run(deg_flat, tab)
    return out.reshape(B, N, D)


# speedup vs baseline: 5.5523x; 1.1036x over previous
import functools

import jax
import jax.numpy as jnp
from jax import lax
from jax.experimental import pallas as pl
from jax.experimental.pallas import tpu as pltpu
from jax.experimental.pallas import tpu_sc as plsc

MAX_DEG = 512
ROWS = 2 * (MAX_DEG + 1)
D = 64
B = 64
N = 2048
P = B * N
NC, NS = 2, 16
NW = NC * NS
PW = P // NW
C = 512
G = C // 16
NCHUNK = PW // C


def _body(deg_hbm, tab_hbm, out_hbm, tab_v, idx0_v, idx1_v, out_v):
    wid = lax.axis_index("s") * NC + lax.axis_index("c")
    pltpu.sync_copy(tab_hbm, tab_v)
    base = wid * PW
    lane = lax.iota(jnp.int32, 16)
    lane64 = lane * D
    rot = [lax.bitwise_and(lane + c, jnp.int32(15)) for c in range(16)]

    def chunk_body(chunk, _):
        off = base + chunk * C
        pltpu.sync_copy(deg_hbm.at[pl.ds(off, C)], idx0_v)
        pltpu.sync_copy(deg_hbm.at[pl.ds(P + off, C)], idx1_v)

        @plsc.parallel_loop(0, G, unroll=2)
        def group(g):
            i0 = idx0_v[pl.ds(g * 16, 16)]
            i1 = idx1_v[pl.ds(g * 16, 16)]
            a0 = jnp.clip(i0, 0, MAX_DEG) * D
            a1 = (jnp.clip(i1, 0, MAX_DEG) + (MAX_DEG + 1)) * D
            sbase = lane64 + g * (16 * D)
            for c in range(16):
                a0c = a0 + rot[c]
                a1c = a1 + rot[c]
                sbc = sbase + rot[c]
                for k in range(0, D, 16):
                    tk = tab_v.at[pl.ds(k, ROWS * D - 48)]
                    ok = out_v.at[pl.ds(k, C * D - 48)]
                    v0 = plsc.load_gather(tk, [a0c])
                    v1 = plsc.load_gather(tk, [a1c])
                    plsc.store_scatter(ok, [sbc], v0 + v1)

        pltpu.sync_copy(out_v, out_hbm.at[pl.ds(off * D, C * D)])
        return 0

    lax.fori_loop(0, NCHUNK, chunk_body, 0, unroll=False)


@jax.jit
def _run(deg_flat, tab_cat):
    mesh = plsc.VectorSubcoreMesh(core_axis_name="c", subcore_axis_name="s")
    f = pl.kernel(
        _body,
        out_type=jax.ShapeDtypeStruct((P * D,), jnp.float32),
        mesh=mesh,
        scratch_types=[
            pltpu.VMEM((ROWS * D,), jnp.float32),
            pltpu.VMEM((C,), jnp.int32),
            pltpu.VMEM((C,), jnp.int32),
            pltpu.VMEM((C * D,), jnp.float32),
        ],
        compiler_params=pltpu.CompilerParams(needs_layout_passes=False),
    )
    return f(deg_flat, tab_cat)


def kernel(degrees, table1, table2):
    tab = jnp.concatenate(
        [table1.at[0].set(0.0), table2.at[0].set(0.0)], axis=0
    ).reshape(ROWS * D)
    deg_flat = degrees.reshape(2 * P)
    out = _run(deg_flat, tab)
    return out.reshape(B, N, D)
